# baseline (speedup 1.0000x reference)
import functools

import jax
import jax.numpy as jnp
from jax.experimental import pallas as pl
from jax.experimental.pallas import tpu as pltpu

_EPS = 1e-5
_R = 512


def _tc_body(ids_ref, frame_ref, table_ref, w_ref, b_ref, out_ref, pos_scr):
    base = pl.program_id(0) * _R

    def gather_one(j, carry):
        pos_scr[j] = table_ref[ids_ref[base + j]]
        return carry

    jax.lax.fori_loop(0, _R, gather_one, 0, unroll=8)

    emb = frame_ref[...] + pos_scr[...]
    mean = jnp.mean(emb, axis=(1, 2), keepdims=True)
    cent = emb - mean
    var = jnp.mean(cent * cent, axis=(1, 2), keepdims=True)
    normed = cent * jax.lax.rsqrt(var + _EPS)
    out_ref[...] = normed * w_ref[...] + b_ref[...]


def kernel(frame_feat, position_ids, pos_table, ln_weight, ln_bias):
    B, S, H = frame_feat.shape
    V = pos_table.shape[0]
    N = B * S
    assert H % 128 == 0
    sub = H // 128

    ids = position_ids.reshape(N).astype(jnp.int32)
    frame_r = frame_feat.reshape(N, sub, 128)
    table_r = pos_table.reshape(V, sub, 128)
    w_r = ln_weight.reshape(sub, 128)
    b_r = ln_bias.reshape(sub, 128)

    grid_spec = pltpu.PrefetchScalarGridSpec(
        num_scalar_prefetch=1,
        grid=(N // _R,),
        in_specs=[
            pl.BlockSpec((_R, sub, 128), lambda i, ids: (i, 0, 0)),
            pl.BlockSpec((V, sub, 128), lambda i, ids: (0, 0, 0)),
            pl.BlockSpec((sub, 128), lambda i, ids: (0, 0)),
            pl.BlockSpec((sub, 128), lambda i, ids: (0, 0)),
        ],
        out_specs=pl.BlockSpec((_R, sub, 128), lambda i, ids: (i, 0, 0)),
        scratch_shapes=[pltpu.VMEM((_R, sub, 128), jnp.float32)],
    )

    out = pl.pallas_call(
        _tc_body,
        grid_spec=grid_spec,
        out_shape=jax.ShapeDtypeStruct((N, sub, 128), jnp.float32),
    )(ids, frame_r, table_r, w_r, b_r)
    return out.reshape(B, S, H)


# speedup vs baseline: 2.6392x; 2.6392x over previous
import functools

import jax
import jax.numpy as jnp
from jax.experimental import pallas as pl
from jax.experimental.pallas import tpu as pltpu

_EPS = 1e-5
_R = 512


def _tc_body(ids_ref, frame_ref, table_ref, w_ref, b_ref, out_ref, pos_scr):
    base = pl.program_id(0) * _R

    def gather_one(j, carry):
        pos_scr[j] = table_ref[ids_ref[base + j]]
        return carry

    jax.lax.fori_loop(0, _R, gather_one, 0, unroll=8)

    emb = frame_ref[...] + pos_scr[...]
    mean = jnp.mean(emb, axis=1, keepdims=True)
    cent = emb - mean
    var = jnp.mean(cent * cent, axis=1, keepdims=True)
    normed = cent * jax.lax.rsqrt(var + _EPS)
    out_ref[...] = normed * w_ref[...] + b_ref[...]


def kernel(frame_feat, position_ids, pos_table, ln_weight, ln_bias):
    B, S, H = frame_feat.shape
    V = pos_table.shape[0]
    N = B * S

    ids = position_ids.reshape(N).astype(jnp.int32)
    frame_r = frame_feat.reshape(N, H)
    w_r = ln_weight.reshape(1, H)
    b_r = ln_bias.reshape(1, H)

    grid_spec = pltpu.PrefetchScalarGridSpec(
        num_scalar_prefetch=1,
        grid=(N // _R,),
        in_specs=[
            pl.BlockSpec((_R, H), lambda i, ids: (i, 0)),
            pl.BlockSpec((V, H), lambda i, ids: (0, 0)),
            pl.BlockSpec((1, H), lambda i, ids: (0, 0)),
            pl.BlockSpec((1, H), lambda i, ids: (0, 0)),
        ],
        out_specs=pl.BlockSpec((_R, H), lambda i, ids: (i, 0)),
        scratch_shapes=[pltpu.VMEM((_R, H), jnp.float32)],
    )

    out = pl.pallas_call(
        _tc_body,
        grid_spec=grid_spec,
        out_shape=jax.ShapeDtypeStruct((N, H), jnp.float32),
    )(ids, frame_r, pos_table, w_r, b_r)
    return out.reshape(B, S, H)
